# Initial kernel scaffold; baseline (speedup 1.0000x reference)
import functools

import jax
import jax.numpy as jnp
import numpy as np
from jax import lax
from jax.experimental import pallas as pl
from jax.experimental.pallas import tpu as pltpu
from jax.experimental.pallas import tpu_sc as plsc

B = 16384
N_OTHER = 30
N_TAGS = 20
N_PUBS = 5
TAG_VOCAB = 100000
PUB_VOCAB = 100000
TAG_DIM = 128
PUB_DIM = 32

NC = 2
NS = 16
NW = NC * NS
SPW = B // NW
CT = 32
NCHUNK = SPW // CT
TROWS = CT * N_TAGS
PROWS = CT * N_PUBS

W_DELTA = np.float32((0.1 - 1.0) / (TAG_VOCAB - 1))


def _pool_body(tags_hbm, pubs_hbm, tag_table, pub_table,
               tag_out, pub_out,
               tidx_v, pidx_v, trows_v, prows_v, tout_v, pout_v, sem):
  cid = lax.axis_index("c")
  sid = lax.axis_index("s")
  wid = sid * NC + cid
  base = wid * SPW

  def chunk_body(ci, _):
    s0 = base + ci * CT
    pltpu.sync_copy(tags_hbm.at[pl.ds(s0 * N_TAGS, TROWS)], tidx_v)
    pltpu.sync_copy(pubs_hbm.at[pl.ds(s0 * N_PUBS, PROWS)], pidx_v)
    cps = []
    for j in range(TROWS // 128):
      cps.append(pltpu.async_copy(
          tag_table.at[tidx_v.at[pl.ds(j * 128, 128)]],
          trows_v.at[pl.ds(j * 128, 128)], sem))
    cps.append(pltpu.async_copy(
        pub_table.at[pidx_v.at[pl.ds(0, 128)]],
        prows_v.at[pl.ds(0, 128)], sem))
    cps.append(pltpu.async_copy(
        pub_table.at[pidx_v.at[pl.ds(128, PROWS - 128)]],
        prows_v.at[pl.ds(128, PROWS - 128)], sem))
    for cp in cps:
      cp.wait()

    def sample_body(si, _):
      tacc = [jnp.zeros((16,), jnp.float32) for _ in range(TAG_DIM // 16)]
      tn = jnp.int32(0)
      for t in range(N_TAGS):
        tag = tidx_v[si * N_TAGS + t]
        m = tag != 0
        w = jnp.where(m, jnp.float32(1.0) + tag.astype(jnp.float32) * W_DELTA,
                      jnp.float32(0.0))
        tn = tn + m.astype(jnp.int32)
        row = si * N_TAGS + t
        for k in range(TAG_DIM // 16):
          tacc[k] = tacc[k] + trows_v[row, pl.ds(k * 16, 16)] * w
      tden = jnp.maximum(tn, 1).astype(jnp.float32)
      for k in range(TAG_DIM // 16):
        tout_v[si, pl.ds(k * 16, 16)] = tacc[k] / tden
      pacc = [jnp.zeros((16,), jnp.float32) for _ in range(PUB_DIM // 16)]
      pn = jnp.int32(0)
      for t in range(N_PUBS):
        pub = pidx_v[si * N_PUBS + t]
        m = pub != 0
        w = jnp.where(m, jnp.float32(1.0), jnp.float32(0.0))
        pn = pn + m.astype(jnp.int32)
        row = si * N_PUBS + t
        for k in range(PUB_DIM // 16):
          pacc[k] = pacc[k] + prows_v[row, pl.ds(k * 16, 16)] * w
      pden = jnp.maximum(pn, 1).astype(jnp.float32)
      for k in range(PUB_DIM // 16):
        pout_v[si, pl.ds(k * 16, 16)] = pacc[k] / pden
      return 0

    lax.fori_loop(0, CT, sample_body, 0)
    pltpu.sync_copy(tout_v, tag_out.at[pl.ds(s0, CT)])
    pltpu.sync_copy(pout_v, pub_out.at[pl.ds(s0, CT)])
    return 0

  lax.fori_loop(0, NCHUNK, chunk_body, 0)


@jax.jit
def _pool(tags, pubs, tag_table, pub_table):
  mesh = plsc.VectorSubcoreMesh(core_axis_name="c", subcore_axis_name="s")
  return pl.kernel(
      _pool_body,
      out_type=[jax.ShapeDtypeStruct((B, TAG_DIM), jnp.float32),
                jax.ShapeDtypeStruct((B, PUB_DIM), jnp.float32)],
      mesh=mesh,
      scratch_types=[
          pltpu.VMEM((TROWS,), jnp.int32),
          pltpu.VMEM((PROWS,), jnp.int32),
          pltpu.VMEM((TROWS, TAG_DIM), jnp.float32),
          pltpu.VMEM((PROWS, PUB_DIM), jnp.float32),
          pltpu.VMEM((CT, TAG_DIM), jnp.float32),
          pltpu.VMEM((CT, PUB_DIM), jnp.float32),
          pltpu.SemaphoreType.DMA,
      ],
  )(tags, pubs, tag_table, pub_table)


MLP_BLK = 2048


def _mlp_body(other_ref, tag_ref, pub_ref,
              W1a, W1b, W1c, b1, W2, b2, W3, b3, W4, b4, W5, b5, out_ref):
  h = (jnp.dot(other_ref[...], W1a[...], preferred_element_type=jnp.float32)
       + jnp.dot(tag_ref[...], W1b[...], preferred_element_type=jnp.float32)
       + jnp.dot(pub_ref[...], W1c[...], preferred_element_type=jnp.float32)
       + b1[...])
  h = jnp.maximum(h, 0.0)
  h = jnp.maximum(jnp.dot(h, W2[...], preferred_element_type=jnp.float32) + b2[...], 0.0)
  h = jnp.maximum(jnp.dot(h, W3[...], preferred_element_type=jnp.float32) + b3[...], 0.0)
  h = jnp.maximum(jnp.dot(h, W4[...], preferred_element_type=jnp.float32) + b4[...], 0.0)
  z = jnp.dot(h, W5[...], preferred_element_type=jnp.float32) + b5[...]
  out_ref[...] = jax.nn.sigmoid(z)


@jax.jit
def _mlp(other, tag_emb, pub_emb, W1, b1, W2, b2, W3, b3, W4, b4, W5, b5):
  nblk = B // MLP_BLK
  bspec = lambda d: pl.BlockSpec((MLP_BLK, d), lambda i: (i, 0))
  wspec = lambda r, c: pl.BlockSpec((r, c), lambda i: (0, 0))
  return pl.pallas_call(
      _mlp_body,
      grid=(nblk,),
      in_specs=[
          bspec(N_OTHER), bspec(TAG_DIM), bspec(PUB_DIM),
          wspec(N_OTHER, 256), wspec(TAG_DIM, 256), wspec(PUB_DIM, 256),
          pl.BlockSpec((256,), lambda i: (0,)),
          wspec(256, 128), pl.BlockSpec((128,), lambda i: (0,)),
          wspec(128, 64), pl.BlockSpec((64,), lambda i: (0,)),
          wspec(64, 32), pl.BlockSpec((32,), lambda i: (0,)),
          wspec(32, 1), pl.BlockSpec((1,), lambda i: (0,)),
      ],
      out_specs=pl.BlockSpec((MLP_BLK, 1), lambda i: (i, 0)),
      out_shape=jax.ShapeDtypeStruct((B, 1), jnp.float32),
  )(other, tag_emb, pub_emb,
    W1[:N_OTHER], W1[N_OTHER:N_OTHER + TAG_DIM], W1[N_OTHER + TAG_DIM:], b1,
    W2, b2, W3, b3, W4, b4, W5, b5)


def kernel(x, tag_table, pub_table, W1, b1, W2, b2, W3, b3, W4, b4, W5, b5):
  other = x[:, :N_OTHER].astype(jnp.float32)
  tags = x[:, N_OTHER:N_OTHER + N_TAGS].astype(jnp.int32).reshape(-1)
  pubs = x[:, N_OTHER + N_TAGS:].astype(jnp.int32).reshape(-1)
  tag_emb, pub_emb = _pool(tags, pubs, tag_table, pub_table)
  return _mlp(other, tag_emb, pub_emb, W1, b1, W2, b2, W3, b3, W4, b4, W5, b5)


# speedup vs baseline: 11.8576x; 11.8576x over previous
import functools

import jax
import jax.numpy as jnp
import numpy as np
from jax import lax
from jax.experimental import pallas as pl
from jax.experimental.pallas import tpu as pltpu
from jax.experimental.pallas import tpu_sc as plsc

B = 16384
N_OTHER = 30
N_TAGS = 20
N_PUBS = 5
TAG_VOCAB = 100000
PUB_VOCAB = 100000
TAG_DIM = 128
PUB_DIM = 32

NC = 2
NS = 16
NW = NC * NS
SPW = B // NW
CT = 32
NCHUNK = SPW // CT
TROWS = CT * N_TAGS
PROWS = CT * N_PUBS

W_DELTA = np.float32((0.1 - 1.0) / (TAG_VOCAB - 1))


def _pool_body(tags_hbm, pubs_hbm, tag_table, pub_table,
               tag_out, pub_out,
               tidx_v, pidx_v, trows_v, prows_v, tout_v, pout_v, sem):
  i32 = jnp.int32
  cid = lax.axis_index("c")
  sid = lax.axis_index("s")
  wid = sid * i32(NC) + cid
  base = wid * i32(SPW)

  def chunk_body(ci, _):
    s0 = base + ci * i32(CT)
    pltpu.sync_copy(tags_hbm.at[pl.ds(s0 * i32(N_TAGS), TROWS)], tidx_v)
    pltpu.sync_copy(pubs_hbm.at[pl.ds(s0 * i32(N_PUBS), PROWS)],
                    pidx_v.at[pl.ds(0, PROWS)])
    cps = []
    for j in range(TROWS // 128):
      cps.append(pltpu.async_copy(
          tag_table.at[tidx_v.at[pl.ds(j * 128, 128)]],
          trows_v.at[pl.ds(j * 128, 128)], sem))
    cps.append(pltpu.async_copy(
        pub_table.at[pidx_v.at[pl.ds(0, 128)]],
        prows_v.at[pl.ds(0, 128)], sem))
    cps.append(pltpu.async_copy(
        pub_table.at[pidx_v.at[pl.ds(128, PROWS - 128)]],
        prows_v.at[pl.ds(128, PROWS - 128)], sem))
    for cp in cps:
      cp.wait()

    def sample_body(si, _):
      tb = si * jnp.int32(N_TAGS)
      va = tidx_v[pl.ds(tb, 16)]
      vb = tidx_v[pl.ds(tb + jnp.int32(4), 16)]
      wa = jnp.where(va != 0, 1.0 + va.astype(jnp.float32) * W_DELTA, 0.0)
      wb = jnp.where(vb != 0, 1.0 + vb.astype(jnp.float32) * W_DELTA, 0.0)
      tacc = [jnp.zeros((16,), jnp.float32) for _ in range(TAG_DIM // 16)]
      tn = jnp.float32(0)
      for t in range(N_TAGS):
        w = wa[t] if t < 16 else wb[t - 4]
        tn = tn + jnp.where(w != 0, jnp.float32(1.0), jnp.float32(0.0))
        row = tb + jnp.int32(t)
        for k in range(TAG_DIM // 16):
          tacc[k] = tacc[k] + trows_v[row, pl.ds(k * 16, 16)] * w
      tden = jnp.maximum(tn, 1.0)
      for k in range(TAG_DIM // 16):
        tout_v[si, pl.ds(k * 16, 16)] = tacc[k] / tden
      pb = si * jnp.int32(N_PUBS)
      pv = pidx_v[pl.ds(pb, 16)]
      pw = jnp.where(pv != 0, jnp.float32(1.0), jnp.float32(0.0))
      pacc = [jnp.zeros((16,), jnp.float32) for _ in range(PUB_DIM // 16)]
      pn = jnp.float32(0)
      for t in range(N_PUBS):
        w = pw[t]
        pn = pn + w
        row = pb + jnp.int32(t)
        for k in range(PUB_DIM // 16):
          pacc[k] = pacc[k] + prows_v[row, pl.ds(k * 16, 16)] * w
      pden = jnp.maximum(pn, 1.0)
      for k in range(PUB_DIM // 16):
        pout_v[si, pl.ds(k * 16, 16)] = pacc[k] / pden
      return jnp.int32(0)

    lax.fori_loop(i32(0), i32(CT), sample_body, i32(0))
    pltpu.sync_copy(tout_v, tag_out.at[pl.ds(s0, CT)])
    pltpu.sync_copy(pout_v, pub_out.at[pl.ds(s0, CT)])
    return i32(0)

  lax.fori_loop(i32(0), i32(NCHUNK), chunk_body, i32(0))


@jax.jit
def _pool(tags, pubs, tag_table, pub_table):
  mesh = plsc.VectorSubcoreMesh(core_axis_name="c", subcore_axis_name="s")
  return pl.kernel(
      _pool_body,
      out_type=[jax.ShapeDtypeStruct((B, TAG_DIM), jnp.float32),
                jax.ShapeDtypeStruct((B, PUB_DIM), jnp.float32)],
      mesh=mesh,
      compiler_params=pltpu.CompilerParams(use_tc_tiling_on_sc=False),
      scratch_types=[
          pltpu.VMEM((TROWS,), jnp.int32),
          pltpu.VMEM((PROWS + 16,), jnp.int32),
          pltpu.VMEM((TROWS, TAG_DIM), jnp.float32),
          pltpu.VMEM((PROWS, PUB_DIM), jnp.float32),
          pltpu.VMEM((CT, TAG_DIM), jnp.float32),
          pltpu.VMEM((CT, PUB_DIM), jnp.float32),
          pltpu.SemaphoreType.DMA,
      ],
  )(tags, pubs, tag_table, pub_table)


MLP_BLK = 2048


def _dot(a, b):
  return jax.lax.dot(a, b, precision=jax.lax.Precision.DEFAULT,
                     preferred_element_type=jnp.float32)


def _mlp_body(other_ref, tag_ref, pub_ref,
              W1, b1, W2, b2, W3, b3, W4, b4, W5, b5, out_ref):
  hin = jnp.concatenate([other_ref[...], tag_ref[...], pub_ref[...]], axis=1)
  h = _dot(hin, W1[...]) + b1[...]
  h = jnp.maximum(h, 0.0)
  h = jnp.maximum(_dot(h, W2[...]) + b2[...], 0.0)
  h = jnp.maximum(_dot(h, W3[...]) + b3[...], 0.0)
  h = jnp.maximum(_dot(h, W4[...]) + b4[...], 0.0)
  z = _dot(h, W5[...]) + b5[...]
  out_ref[...] = jax.nn.sigmoid(z)


@jax.jit
def _mlp(other, tag_emb, pub_emb, W1, b1, W2, b2, W3, b3, W4, b4, W5, b5):
  nblk = B // MLP_BLK
  z = np.int32(0)
  bspec = lambda d: pl.BlockSpec((MLP_BLK, d), lambda i: (i, z))
  wspec = lambda r, c: pl.BlockSpec((r, c), lambda i: (z, z))
  vspec = lambda d: pl.BlockSpec((d,), lambda i: (z,))
  return pl.pallas_call(
      _mlp_body,
      grid=(nblk,),
      in_specs=[
          bspec(N_OTHER), bspec(TAG_DIM), bspec(PUB_DIM),
          wspec(N_OTHER + TAG_DIM + PUB_DIM, 256),
          vspec(256),
          wspec(256, 128), vspec(128),
          wspec(128, 64), vspec(64),
          wspec(64, 32), vspec(32),
          wspec(32, 1), vspec(1),
      ],
      out_specs=pl.BlockSpec((MLP_BLK, 1), lambda i: (i, z)),
      out_shape=jax.ShapeDtypeStruct((B, 1), jnp.float32),
  )(other, tag_emb, pub_emb, W1, b1, W2, b2, W3, b3, W4, b4, W5, b5)


def kernel(x, tag_table, pub_table, W1, b1, W2, b2, W3, b3, W4, b4, W5, b5):
  other = x[:, :N_OTHER].astype(jnp.float32)
  tags = x[:, N_OTHER:N_OTHER + N_TAGS].astype(jnp.int32).reshape(-1)
  pubs = x[:, N_OTHER + N_TAGS:].astype(jnp.int32).reshape(-1)
  tag_emb, pub_emb = _pool(tags, pubs, tag_table, pub_table)
  return _mlp(other, tag_emb, pub_emb, W1, b1, W2, b2, W3, b3, W4, b4, W5, b5)


# speedup vs baseline: 12.9651x; 1.0934x over previous
import functools

import jax
import jax.numpy as jnp
import numpy as np
from jax import lax
from jax.experimental import pallas as pl
from jax.experimental.pallas import tpu as pltpu
from jax.experimental.pallas import tpu_sc as plsc

B = 16384
N_OTHER = 30
N_TAGS = 20
N_PUBS = 5
TAG_VOCAB = 100000
PUB_VOCAB = 100000
TAG_DIM = 128
PUB_DIM = 32

NC = 2
NS = 16
NW = NC * NS
SPW = B // NW
CT = 32
NCHUNK = SPW // CT
TROWS = CT * N_TAGS
PROWS = CT * N_PUBS

W_DELTA = np.float32((0.1 - 1.0) / (TAG_VOCAB - 1))


def _pool_body(x32_hbm, tag_table, pub_table,
               tag_out, pub_out, oth_out,
               xi_v, tidx_v, pidx_v, trows_v, prows_v,
               tout_v, pout_v, oout_v, sem):
  i32 = jnp.int32
  cid = lax.axis_index("c")
  sid = lax.axis_index("s")
  wid = sid * i32(NC) + cid
  base = wid * i32(SPW)
  lanes = lax.iota(jnp.int32, 16)
  col_t0 = 60 + 2 * lanes
  col_t1 = jnp.where(lanes < 4, 92 + 2 * lanes, 92)
  col_p = jnp.where(lanes < 5, 100 + 2 * lanes, 108)
  col_o0 = 2 * lanes
  col_o1 = jnp.where(lanes < 14, 32 + 2 * lanes, 58)

  def chunk_body(ci, _):
    s0 = base + ci * i32(CT)
    pltpu.sync_copy(x32_hbm.at[pl.ds(s0, CT)], xi_v)

    def extract_body(si, _):
      tb = si * i32(N_TAGS)
      pb = si * i32(N_PUBS)
      ob = si * i32(N_OTHER)
      row = jnp.full((16,), si, jnp.int32)
      tg0 = plsc.load_gather(xi_v, [row, col_t0])
      tg1 = plsc.load_gather(xi_v, [row, col_t1])
      pg = plsc.load_gather(xi_v, [row, col_p])
      og0 = plsc.load_gather(xi_v, [row, col_o0]).astype(jnp.float32)
      og1 = plsc.load_gather(xi_v, [row, col_o1]).astype(jnp.float32)
      tidx_v[pl.ds(tb, 16)] = tg0
      tidx_v[pl.ds(tb + i32(16), 16)] = tg1
      pidx_v[pl.ds(pb, 16)] = pg
      oout_v[pl.ds(ob, 16)] = og0
      oout_v[pl.ds(ob + i32(16), 16)] = og1
      return i32(0)

    lax.fori_loop(i32(0), i32(CT), extract_body, i32(0))
    cps = []
    for j in range(TROWS // 128):
      cps.append(pltpu.async_copy(
          tag_table.at[tidx_v.at[pl.ds(j * 128, 128)]],
          trows_v.at[pl.ds(j * 128, 128)], sem))
    cps.append(pltpu.async_copy(
        pub_table.at[pidx_v.at[pl.ds(0, 128)]],
        prows_v.at[pl.ds(0, 128)], sem))
    cps.append(pltpu.async_copy(
        pub_table.at[pidx_v.at[pl.ds(128, PROWS - 128)]],
        prows_v.at[pl.ds(128, PROWS - 128)], sem))
    for cp in cps:
      cp.wait()

    def sample_body(si, _):
      tb = si * jnp.int32(N_TAGS)
      va = tidx_v[pl.ds(tb, 16)]
      vb = tidx_v[pl.ds(tb + jnp.int32(4), 16)]
      wa = jnp.where(va != 0, 1.0 + va.astype(jnp.float32) * W_DELTA, 0.0)
      wb = jnp.where(vb != 0, 1.0 + vb.astype(jnp.float32) * W_DELTA, 0.0)
      tacc = [jnp.zeros((16,), jnp.float32) for _ in range(TAG_DIM // 16)]
      tn = jnp.float32(0)
      for t in range(N_TAGS):
        w = wa[t] if t < 16 else wb[t - 4]
        tn = tn + jnp.where(w != 0, jnp.float32(1.0), jnp.float32(0.0))
        row = tb + jnp.int32(t)
        for k in range(TAG_DIM // 16):
          tacc[k] = tacc[k] + trows_v[row, pl.ds(k * 16, 16)] * w
      tden = jnp.maximum(tn, 1.0)
      for k in range(TAG_DIM // 16):
        tout_v[si, pl.ds(k * 16, 16)] = tacc[k] / tden
      pb = si * jnp.int32(N_PUBS)
      pv = pidx_v[pl.ds(pb, 16)]
      pw = jnp.where(pv != 0, jnp.float32(1.0), jnp.float32(0.0))
      pacc = [jnp.zeros((16,), jnp.float32) for _ in range(PUB_DIM // 16)]
      pn = jnp.float32(0)
      for t in range(N_PUBS):
        w = pw[t]
        pn = pn + w
        row = pb + jnp.int32(t)
        for k in range(PUB_DIM // 16):
          pacc[k] = pacc[k] + prows_v[row, pl.ds(k * 16, 16)] * w
      pden = jnp.maximum(pn, 1.0)
      for k in range(PUB_DIM // 16):
        pout_v[si, pl.ds(k * 16, 16)] = pacc[k] / pden
      return jnp.int32(0)

    lax.fori_loop(i32(0), i32(CT), sample_body, i32(0))
    pltpu.sync_copy(tout_v, tag_out.at[pl.ds(s0, CT)])
    pltpu.sync_copy(pout_v, pub_out.at[pl.ds(s0, CT)])
    pltpu.sync_copy(oout_v.at[pl.ds(0, CT * N_OTHER)],
                    oth_out.at[pl.ds(s0 * i32(N_OTHER), CT * N_OTHER)])
    return i32(0)

  lax.fori_loop(i32(0), i32(NCHUNK), chunk_body, i32(0))


@jax.jit
def _pool(x32, tag_table, pub_table):
  mesh = plsc.VectorSubcoreMesh(core_axis_name="c", subcore_axis_name="s")
  return pl.kernel(
      _pool_body,
      out_type=[jax.ShapeDtypeStruct((B, TAG_DIM), jnp.float32),
                jax.ShapeDtypeStruct((B, PUB_DIM), jnp.float32),
                jax.ShapeDtypeStruct((B * N_OTHER,), jnp.float32)],
      mesh=mesh,
      compiler_params=pltpu.CompilerParams(use_tc_tiling_on_sc=False,
                                           needs_layout_passes=False),
      scratch_types=[
          pltpu.VMEM((CT, 2 * (N_OTHER + N_TAGS + N_PUBS)), jnp.int32),
          pltpu.VMEM((TROWS + 16,), jnp.int32),
          pltpu.VMEM((PROWS + 16,), jnp.int32),
          pltpu.VMEM((TROWS, TAG_DIM), jnp.float32),
          pltpu.VMEM((PROWS, PUB_DIM), jnp.float32),
          pltpu.VMEM((CT, TAG_DIM), jnp.float32),
          pltpu.VMEM((CT, PUB_DIM), jnp.float32),
          pltpu.VMEM((CT * N_OTHER + 16,), jnp.float32),
          pltpu.SemaphoreType.DMA,
      ],
  )(x32, tag_table, pub_table)


MLP_BLK = 2048


def _dot(a, b):
  return jax.lax.dot(a, b, precision=jax.lax.Precision.DEFAULT,
                     preferred_element_type=jnp.float32)


def _mlp_body(other_ref, tag_ref, pub_ref,
              W1, b1, W2, b2, W3, b3, W4, b4, W5, b5, out_ref):
  hin = jnp.concatenate([other_ref[...], tag_ref[...], pub_ref[...]], axis=1)
  h = _dot(hin, W1[...]) + b1[...]
  h = jnp.maximum(h, 0.0)
  h = jnp.maximum(_dot(h, W2[...]) + b2[...], 0.0)
  h = jnp.maximum(_dot(h, W3[...]) + b3[...], 0.0)
  h = jnp.maximum(_dot(h, W4[...]) + b4[...], 0.0)
  z = _dot(h, W5[...]) + b5[...]
  out_ref[...] = jax.nn.sigmoid(z)


@jax.jit
def _mlp(other, tag_emb, pub_emb, W1, b1, W2, b2, W3, b3, W4, b4, W5, b5):
  nblk = B // MLP_BLK
  z = np.int32(0)
  bspec = lambda d: pl.BlockSpec((MLP_BLK, d), lambda i: (i, z))
  wspec = lambda r, c: pl.BlockSpec((r, c), lambda i: (z, z))
  vspec = lambda d: pl.BlockSpec((d,), lambda i: (z,))
  return pl.pallas_call(
      _mlp_body,
      grid=(nblk,),
      in_specs=[
          bspec(N_OTHER), bspec(TAG_DIM), bspec(PUB_DIM),
          wspec(N_OTHER + TAG_DIM + PUB_DIM, 256),
          vspec(256),
          wspec(256, 128), vspec(128),
          wspec(128, 64), vspec(64),
          wspec(64, 32), vspec(32),
          wspec(32, 1), vspec(1),
      ],
      out_specs=pl.BlockSpec((MLP_BLK, 1), lambda i: (i, z)),
      out_shape=jax.ShapeDtypeStruct((B, 1), jnp.float32),
  )(other, tag_emb, pub_emb, W1, b1, W2, b2, W3, b3, W4, b4, W5, b5)


def kernel(x, tag_table, pub_table, W1, b1, W2, b2, W3, b3, W4, b4, W5, b5):
  x32 = jax.lax.bitcast_convert_type(x, jnp.int32).reshape(B, -1)
  tag_emb, pub_emb, other_flat = _pool(x32, tag_table, pub_table)
  other = other_flat.reshape(B, N_OTHER)
  return _mlp(other, tag_emb, pub_emb, W1, b1, W2, b2, W3, b3, W4, b4, W5, b5)
